# baseline (speedup 1.0000x reference)
import functools

import jax
import jax.numpy as jnp
from jax import lax
from jax.experimental import pallas as pl
from jax.experimental.pallas import tpu as pltpu
from jax.experimental.pallas import tpu_sc as plsc

NOBJ = 1024
NREL = 4096
OBJ_DIM = 4096
H = 512
NCLS = 151
NRC = 51
NRCP = 128

_SC_NC = 2
_SC_NS = 16
_SC_NW = _SC_NC * _SC_NS
_BPW = NREL // _SC_NW
_CH = 32
_NCH = _BPW // _CH

_BF = jnp.bfloat16


_SC_MESH = plsc.VectorSubcoreMesh(core_axis_name="c", subcore_axis_name="s")


def _sc_front(boxp, fa, fb, g, idx_a, idx_b):
    CH = 64
    NSTREAM = 4

    @functools.partial(
        pl.kernel, mesh=_SC_MESH,
        out_type=(jax.ShapeDtypeStruct((NREL, 128), jnp.float32),
                  jax.ShapeDtypeStruct((NREL, 128), jnp.float32),
                  jax.ShapeDtypeStruct((NREL, H), jnp.float32),
                  jax.ShapeDtypeStruct((NREL, H), jnp.float32),
                  jax.ShapeDtypeStruct((NREL, H), jnp.float32),
                  jax.ShapeDtypeStruct((NREL, H), jnp.float32)),
        scratch_types=[
            pltpu.VMEM((_BPW,), jnp.int32),
            pltpu.VMEM((_BPW,), jnp.int32),
            pltpu.VMEM((64, 128), jnp.float32),
            pltpu.VMEM((CH, H), jnp.float32),
            pltpu.VMEM((CH, H), jnp.float32),
            pltpu.VMEM((CH, H), jnp.float32),
        ] + [pltpu.SemaphoreType.DMA] * 8,
    )
    def k(boxp_h, fa_h, fb_h, g_h, ia_h, ib_h,
          bs_h, bo_h, e1a_h, e1b_h, gpa_h, gpb_h,
          ia_v, ib_v, rbox_v, v0, v1, v2,
          g0, g1, g2, w0, w1, w2, s4, s5):
        base = _worker_base()
        pltpu.sync_copy(ia_h.at[pl.ds(base, _BPW)], ia_v)
        pltpu.sync_copy(ib_h.at[pl.ds(base, _BPW)], ib_v)

        bufs = (v0, v1, v2)
        gsems = (g0, g1, g2)
        wsems = (w0, w1, w2)
        xfers = []
        for tab, use_a, out in ((fa_h, True, e1a_h), (fb_h, False, e1b_h),
                                (g_h, True, gpa_h), (g_h, False, gpb_h)):
            for c in range(_BPW // CH):
                xfers.append((tab, use_a, c, out))

        n = len(xfers)
        gcp = [None] * n
        wcp = [None] * n

        def issue(i):
            tab, use_a, c, _ = xfers[i]
            idx = ia_v if use_a else ib_v
            gcp[i] = pltpu.async_copy(
                tab.at[idx.at[pl.ds(c * CH, CH)]], bufs[i % 3],
                gsems[i % 3])

        issue(0)
        issue(1)
        issue(2)
        for i in range(n):
            _, _, c, out = xfers[i]
            off = base + c * CH
            gcp[i].wait()
            wcp[i] = pltpu.async_copy(bufs[i % 3], out.at[pl.ds(off, CH)],
                                      wsems[i % 3])
            if i + 3 < n:
                wcp[i].wait()
                issue(i + 3)
        wcp[n - 3].wait()
        wcp[n - 2].wait()
        wcp[n - 1].wait()
        for idx_v, boxout in ((ia_v, bs_h), (ib_v, bo_h)):
            for hh in range(2):
                pltpu.async_copy(
                    boxp_h.at[idx_v.at[pl.ds(hh * 64, 64)]], rbox_v,
                    s4).wait()
                pltpu.sync_copy(rbox_v,
                                boxout.at[pl.ds(base + hh * 64, 64)])

    return k(boxp, fa, fb, g, idx_a, idx_b)


def _worker_base():
    wid = lax.axis_index("s") * _SC_NC + lax.axis_index("c")
    return wid * _BPW


def _node_proj_body(obj_fmaps, W_obj, b_obj, W_g1a, W_g1b, cls_embp, labels,
                    f_o, fa_o, fb_o, g_o):
    f = jnp.dot(obj_fmaps[...].astype(_BF), W_obj[...].astype(_BF),
                preferred_element_type=jnp.float32)
    f = f + b_obj[...]
    f_o[...] = f
    f16 = f.astype(_BF)
    fa_o[...] = jnp.dot(f16, W_g1a[...], preferred_element_type=jnp.float32)
    fb_o[...] = jnp.dot(f16, W_g1b[...], preferred_element_type=jnp.float32)
    lab = labels[...]
    oh = (lab == jax.lax.broadcasted_iota(jnp.int32, (NOBJ, 256), 1))
    emb = jnp.dot(oh.astype(jnp.float32), cls_embp[...],
                  preferred_element_type=jnp.float32)
    g_o[...] = emb + f


def _fold_body(W_rel, W_g1c, b_rel, b_g1, Wrc_o, crow_o):
    Wrc_o[...] = jnp.dot(W_rel[...].astype(_BF), W_g1c[...],
                         preferred_element_type=jnp.float32).astype(_BF)
    crow_o[...] = jnp.dot(b_rel[...].astype(jnp.float32),
                          W_g1c[...].astype(jnp.float32),
                          preferred_element_type=jnp.float32) + b_g1[...]


def _box_feats(bsT, boT):
    def row(t, i):
        return t[i:i + 1, :]
    sx1, sy1, sx2, sy2 = (row(bsT, i) for i in range(4))
    ox1, oy1, ox2, oy2 = (row(boT, i) for i in range(4))
    px1 = jnp.minimum(sx1, ox1)
    py1 = jnp.minimum(sy1, oy1)
    px2 = jnp.maximum(sx2, ox2)
    py2 = jnp.maximum(sy2, oy2)

    def ctr(x1, y1, x2, y2):
        return ((x1 + x2) * 0.5, (y1 + y2) * 0.5,
                (x2 - x1) * 0.5, (y2 - y1) * 0.5)

    scx, scy, sw, sh = ctr(sx1, sy1, sx2, sy2)
    ocx, ocy, ow, oh = ctr(ox1, oy1, ox2, oy2)
    pcx, pcy, pw, ph = ctr(px1, py1, px2, py2)

    def delta(a, b):
        (acx, acy, aw, ah), (bcx, bcy, bw, bh) = a, b
        return [(acx - bcx) / bw, (acy - bcy) / bh,
                jnp.log(aw / bw), jnp.log(ah * bh)]

    def c5(x1, y1, x2, y2):
        return [x1 / 592.0, y1 / 592.0, (x1 + x2) / 592.0,
                (y1 + y2) / 592.0, x2 * y2 / (592.0 ** 2)]

    rows = []
    rows += delta((scx, scy, sw, sh), (ocx, ocy, ow, oh))
    rows += delta((scx, scy, sw, sh), (pcx, pcy, pw, ph))
    rows += delta((pcx, pcy, pw, ph), (ocx, ocy, ow, oh))
    rows += c5(sx1, sy1, sx2, sy2)
    rows += c5(ox1, oy1, ox2, oy2)
    return rows


def _edge_e_body(vr, Wrc, e1a, e1b, bsT, boT, W_boxp, crow, e_o):
    vc = jnp.dot(vr[...].astype(_BF), Wrc[...],
                 preferred_element_type=jnp.float32)
    rows = _box_feats(bsT[...], boT[...])
    bfT = jnp.concatenate(rows + [jnp.zeros_like(rows[0])] * 10, axis=0)
    bfW = jax.lax.dot_general(bfT, W_boxp[...], (((0,), (0,)), ((), ())),
                              preferred_element_type=jnp.float32)
    e_o[...] = jax.nn.relu(vc + e1a[...] + e1b[...] + bfW + crow[...])


def _seg_node_body(sub_row, obj_row, e, W, out_o, *, bn):
    n0 = pl.program_id(0) * bn
    ids = jax.lax.broadcasted_iota(jnp.int32, (bn, NREL), 0) + n0
    pt = ((ids == sub_row[0:1, :]).astype(_BF)
          + (ids == obj_row[0:1, :]).astype(_BF))
    agg = jnp.dot(pt, e[...].astype(_BF), preferred_element_type=jnp.float32)
    out_o[...] = jax.nn.relu(
        jnp.dot(agg.astype(_BF), W[...],
                preferred_element_type=jnp.float32)).astype(_BF)


def _edge_update_body(idx_a, idx_b, node, e, W_outp, e2_o, l_o):
    ia = idx_a[...]
    ib = idx_b[...]
    ids = jax.lax.broadcasted_iota(jnp.int32, (ia.shape[0], NOBJ), 1)
    p = ((ia == ids).astype(_BF) + (ib == ids).astype(_BF))
    np_ = jnp.dot(p, node[...], preferred_element_type=jnp.float32)
    e2 = jax.nn.relu(e[...] + np_)
    e2_o[...] = e2
    l_o[...] = jnp.dot(e2.astype(_BF), W_outp[...],
                       preferred_element_type=jnp.float32)


def _edge_l_body(idx_a, idx_b, node, e, W_outp, l_o):
    ia = idx_a[...]
    ib = idx_b[...]
    ids = jax.lax.broadcasted_iota(jnp.int32, (ia.shape[0], NOBJ), 1)
    p = ((ia == ids).astype(_BF) + (ib == ids).astype(_BF))
    np_ = jnp.dot(p, node[...], preferred_element_type=jnp.float32)
    e3 = jax.nn.relu(e[...] + np_)
    l_o[...] = jnp.dot(e3.astype(_BF), W_outp[...],
                       preferred_element_type=jnp.float32)


def _hh_body(gpa, gpb, bpT, W_bp, W_voutp, l1, lv_o, rel0_o):
    bpW = jax.lax.dot_general(bpT[...], W_bp[...], (((0,), (0,)), ((), ())),
                              preferred_element_type=jnp.float32)
    hh = jax.nn.relu(gpa[...] + gpb[...] + bpW)
    lv = jnp.dot(hh.astype(_BF), W_voutp[...],
                 preferred_element_type=jnp.float32)
    lv_o[...] = lv
    rel0_o[...] = l1[...] + lv


def _f32(shape):
    return jax.ShapeDtypeStruct(shape, jnp.float32)


def _bf16(shape):
    return jax.ShapeDtypeStruct(shape, _BF)


def kernel(obj_fmaps, obj_logits, rel_inds, vr, obj_labels, bboxes,
           obj_logits_fc, W_obj, b_obj, W_rel, b_rel, W_g1, W_box, b_g1,
           W_n1, W_n2, W_out1, W_out2, cls_emb, W_b, W_vout):
    sub = rel_inds[:, 1]
    objn = rel_inds[:, 2]
    sub2d = sub.reshape(NREL, 1)
    obj2d = objn.reshape(NREL, 1)
    sub_row = jnp.broadcast_to(sub[None, :], (8, NREL))
    obj_row = jnp.broadcast_to(objn[None, :], (8, NREL))
    lab2d = obj_labels.reshape(NOBJ, 1)

    W_g1a = W_g1[:H].astype(_BF)
    W_g1b = W_g1[H:2 * H].astype(_BF)
    W_g1c = W_g1[2 * H:].astype(_BF)
    W_n1_16 = W_n1.astype(_BF)
    W_n2_16 = W_n2.astype(_BF)
    cls_embp = jnp.zeros((256, H), jnp.float32).at[:NCLS].set(cls_emb)
    W_boxp = jnp.zeros((32, H), jnp.float32).at[:22].set(W_box)
    W_out1p = jnp.zeros((H, NRCP), _BF).at[:, :NRC].set(W_out1.astype(_BF))
    W_out2p = jnp.zeros((H, NRCP), _BF).at[:, :NRC].set(W_out2.astype(_BF))
    W_voutp = jnp.zeros((H, NRCP), _BF).at[:, :NRC].set(W_vout.astype(_BF))
    W_bp = jnp.zeros((16, H), jnp.float32).at[:8].set(W_b)
    boxp = jnp.zeros((NOBJ, 128), jnp.float32).at[:, :4].set(bboxes)
    b_obj_r = b_obj.reshape(1, H)
    b_rel_r = b_rel.reshape(1, H)
    b_g1_r = b_g1.reshape(1, H)

    f, fa, fb, g = pl.pallas_call(
        _node_proj_body,
        out_shape=(_f32((NOBJ, H)),) * 4,
    )(obj_fmaps, W_obj, b_obj_r, W_g1a, W_g1b, cls_embp, lab2d)

    bsg, bog, e1a, e1b, gpa, gpb = _sc_front(boxp, fa, fb, g, sub, objn)
    bsT = bsg[:, :16].T
    boT = bog[:, :16].T
    bpT = jnp.concatenate([bsT[:4], boT[:4],
                           jnp.zeros((8, NREL), jnp.float32)], 0) / 592.0

    Wrc, crow = pl.pallas_call(
        _fold_body,
        out_shape=(_bf16((OBJ_DIM, H)), _f32((1, H))),
    )(W_rel, W_g1c, b_rel_r, b_g1_r)

    BM = 512
    e = pl.pallas_call(
        _edge_e_body,
        grid=(NREL // BM,),
        in_specs=[pl.BlockSpec((BM, OBJ_DIM), lambda i: (i, 0)),
                  pl.BlockSpec((OBJ_DIM, H), lambda i: (0, 0)),
                  pl.BlockSpec((BM, H), lambda i: (i, 0)),
                  pl.BlockSpec((BM, H), lambda i: (i, 0)),
                  pl.BlockSpec((16, BM), lambda i: (0, i)),
                  pl.BlockSpec((16, BM), lambda i: (0, i)),
                  pl.BlockSpec((32, H), lambda i: (0, 0)),
                  pl.BlockSpec((1, H), lambda i: (0, 0))],
        out_specs=pl.BlockSpec((BM, H), lambda i: (i, 0)),
        out_shape=_f32((NREL, H)),
    )(vr, Wrc, e1a, e1b, bsT, boT, W_boxp, crow)

    BN = 512
    seg_node = pl.pallas_call(
        functools.partial(_seg_node_body, bn=BN),
        grid=(NOBJ // BN,),
        in_specs=[pl.BlockSpec((8, NREL), lambda i: (0, 0)),
                  pl.BlockSpec((8, NREL), lambda i: (0, 0)),
                  pl.BlockSpec((NREL, H), lambda i: (0, 0)),
                  pl.BlockSpec((H, H), lambda i: (0, 0))],
        out_specs=pl.BlockSpec((BN, H), lambda i: (i, 0)),
        out_shape=_bf16((NOBJ, H)),
    )
    node = seg_node(sub_row, obj_row, e, W_n1_16)

    BE = 1024
    edge_update = pl.pallas_call(
        _edge_update_body,
        grid=(NREL // BE,),
        in_specs=[pl.BlockSpec((BE, 1), lambda i: (i, 0)),
                  pl.BlockSpec((BE, 1), lambda i: (i, 0)),
                  pl.BlockSpec((NOBJ, H), lambda i: (0, 0)),
                  pl.BlockSpec((BE, H), lambda i: (i, 0)),
                  pl.BlockSpec((H, NRCP), lambda i: (0, 0))],
        out_specs=(pl.BlockSpec((BE, H), lambda i: (i, 0)),
                   pl.BlockSpec((BE, NRCP), lambda i: (i, 0))),
        out_shape=(_f32((NREL, H)), _f32((NREL, NRCP))),
    )
    e2, l1p = edge_update(sub2d, obj2d, node, e, W_out1p)

    node2 = seg_node(sub_row, obj_row, e2, W_n2_16)

    l2p = pl.pallas_call(
        _edge_l_body,
        grid=(NREL // BE,),
        in_specs=[pl.BlockSpec((BE, 1), lambda i: (i, 0)),
                  pl.BlockSpec((BE, 1), lambda i: (i, 0)),
                  pl.BlockSpec((NOBJ, H), lambda i: (0, 0)),
                  pl.BlockSpec((BE, H), lambda i: (i, 0)),
                  pl.BlockSpec((H, NRCP), lambda i: (0, 0))],
        out_specs=pl.BlockSpec((BE, NRCP), lambda i: (i, 0)),
        out_shape=_f32((NREL, NRCP)),
    )(sub2d, obj2d, node2, e2, W_out2p)

    lvp, rel0p = pl.pallas_call(
        _hh_body,
        grid=(NREL // BE,),
        in_specs=[pl.BlockSpec((BE, H), lambda i: (i, 0)),
                  pl.BlockSpec((BE, H), lambda i: (i, 0)),
                  pl.BlockSpec((16, BE), lambda i: (0, i)),
                  pl.BlockSpec((16, H), lambda i: (0, 0)),
                  pl.BlockSpec((H, NRCP), lambda i: (0, 0)),
                  pl.BlockSpec((BE, NRCP), lambda i: (i, 0))],
        out_specs=(pl.BlockSpec((BE, NRCP), lambda i: (i, 0)),
                   pl.BlockSpec((BE, NRCP), lambda i: (i, 0))),
        out_shape=(_f32((NREL, NRCP)), _f32((NREL, NRCP))),
    )(gpa, gpb, bpT, W_bp, W_voutp, l1p)

    rel0 = rel0p[:, :NRC]
    l2 = l2p[:, :NRC]
    lv = lvp[:, :NRC]
    return (obj_logits, obj_labels, rel0, l2, lv)
